# baseline (speedup 1.0000x reference)
import functools

import jax
import jax.numpy as jnp
from jax import lax
from jax.experimental import pallas as pl
from jax.experimental.pallas import tpu as pltpu
from jax.experimental.pallas import tpu_sc as plsc

_N_CLASSES = 1000
_CTX_LEN = 77
_N_CTX = 4
_D = 768
_SUF = _CTX_LEN - 1 - _N_CTX

_NC = 2
_NS = 16
_NW = _NC * _NS

_BIG_W = 8
_BIG_N = 32
_SMALL_N = 31


def _body(prefix_hbm, ctx_hbm, suffix_hbm, out_hbm, ctx_vmem, sem):
    wid = lax.axis_index("s") * _NC + lax.axis_index("c")
    pltpu.sync_copy(ctx_hbm, ctx_vmem)

    def issue(base, n):
        copies = [
            pltpu.async_copy(
                prefix_hbm.at[pl.ds(base, n)],
                out_hbm.at[pl.ds(base, n), pl.ds(0, 1), :],
                sem,
            ),
            pltpu.async_copy(
                suffix_hbm.at[pl.ds(base, n)],
                out_hbm.at[pl.ds(base, n), pl.ds(_N_CTX + 1, _SUF), :],
                sem,
            ),
        ]
        for i in range(n):
            copies.append(
                pltpu.async_copy(
                    ctx_vmem,
                    out_hbm.at[pl.ds(base + i, 1), pl.ds(1, _N_CTX), :],
                    sem,
                )
            )
        for c in copies:
            c.wait()

    @pl.when(wid < _BIG_W)
    def _():
        issue(wid * _BIG_N, _BIG_N)

    @pl.when(wid >= _BIG_W)
    def _():
        issue(_BIG_W * _BIG_N + (wid - _BIG_W) * _SMALL_N, _SMALL_N)


@jax.jit
def _prompt_concat(token_prefix, ctx_embedding, token_suffix):
    run = functools.partial(
        pl.kernel,
        out_type=jax.ShapeDtypeStruct((_N_CLASSES, _CTX_LEN, _D), jnp.float32),
        mesh=plsc.VectorSubcoreMesh(core_axis_name="c", subcore_axis_name="s"),
        scratch_types=[
            pltpu.VMEM((1, _N_CTX, _D), jnp.float32),
            pltpu.SemaphoreType.DMA,
        ],
        compiler_params=pltpu.CompilerParams(use_tc_tiling_on_sc=False),
    )(_body)
    return run(token_prefix, ctx_embedding.reshape(1, _N_CTX, _D), token_suffix)


def kernel(token_prefix, ctx_embedding, token_suffix, eos_position):
    prompts = _prompt_concat(token_prefix, ctx_embedding, token_suffix)
    return (prompts, eos_position)


# speedup vs baseline: 25.4219x; 25.4219x over previous
import functools

import jax
import jax.numpy as jnp
from jax.experimental import pallas as pl
from jax.experimental.pallas import tpu as pltpu

_N_CLASSES = 1000
_CTX_LEN = 77
_N_CTX = 4
_D = 768
_SUF = _CTX_LEN - 1 - _N_CTX

_CB = 8


def _body(prefix_ref, ctx_ref, suffix_ref, out_ref):
    out_ref[:, 0:1, :] = prefix_ref[...]
    out_ref[:, 1 : 1 + _N_CTX, :] = jnp.broadcast_to(
        ctx_ref[...][None], (_CB, _N_CTX, _D)
    )
    out_ref[:, 1 + _N_CTX :, :] = suffix_ref[...]


@jax.jit
def _prompt_concat(token_prefix, ctx_embedding, token_suffix):
    grid = (_N_CLASSES // _CB,)
    return pl.pallas_call(
        _body,
        grid=grid,
        in_specs=[
            pl.BlockSpec((_CB, 1, _D), lambda i: (i, 0, 0)),
            pl.BlockSpec((_N_CTX, _D), lambda i: (0, 0)),
            pl.BlockSpec((_CB, _SUF, _D), lambda i: (i, 0, 0)),
        ],
        out_specs=pl.BlockSpec((_CB, _CTX_LEN, _D), lambda i: (i, 0, 0)),
        out_shape=jax.ShapeDtypeStruct((_N_CLASSES, _CTX_LEN, _D), jnp.float32),
        compiler_params=pltpu.CompilerParams(
            dimension_semantics=("arbitrary",),
        ),
    )(token_prefix, ctx_embedding, token_suffix)


def kernel(token_prefix, ctx_embedding, token_suffix, eos_position):
    prompts = _prompt_concat(token_prefix, ctx_embedding, token_suffix)
    return (prompts, eos_position)


# speedup vs baseline: 27.0269x; 1.0631x over previous
import functools

import jax
import jax.numpy as jnp
from jax.experimental import pallas as pl
from jax.experimental.pallas import tpu as pltpu

_N_CLASSES = 1000
_CTX_LEN = 77
_N_CTX = 4
_D = 768
_SUF = _CTX_LEN - 1 - _N_CTX

_CB = 40


def _body(prefix_ref, ctx_ref, suffix_ref, out_ref):
    out_ref[:, 0:1, :] = prefix_ref[...]
    out_ref[:, 1 : 1 + _N_CTX, :] = jnp.broadcast_to(
        ctx_ref[...][None], (_CB, _N_CTX, _D)
    )
    out_ref[:, 1 + _N_CTX :, :] = suffix_ref[...]


@jax.jit
def _prompt_concat(token_prefix, ctx_embedding, token_suffix):
    grid = (_N_CLASSES // _CB,)
    return pl.pallas_call(
        _body,
        grid=grid,
        in_specs=[
            pl.BlockSpec((_CB, 1, _D), lambda i: (i, 0, 0)),
            pl.BlockSpec((_N_CTX, _D), lambda i: (0, 0)),
            pl.BlockSpec((_CB, _SUF, _D), lambda i: (i, 0, 0)),
        ],
        out_specs=pl.BlockSpec((_CB, _CTX_LEN, _D), lambda i: (i, 0, 0)),
        out_shape=jax.ShapeDtypeStruct((_N_CLASSES, _CTX_LEN, _D), jnp.float32),
        compiler_params=pltpu.CompilerParams(
            dimension_semantics=("arbitrary",),
        ),
    )(token_prefix, ctx_embedding, token_suffix)


def kernel(token_prefix, ctx_embedding, token_suffix, eos_position):
    prompts = _prompt_concat(token_prefix, ctx_embedding, token_suffix)
    return (prompts, eos_position)
